# baseline (speedup 1.0000x reference)
import functools

import jax
import jax.numpy as jnp
from jax import lax
from jax.experimental import pallas as pl
from jax.experimental.pallas import tpu as pltpu
from jax.experimental.pallas import tpu_sc as plsc

DIM = 64
NUM_WORKERS = 32
CHUNK = 128
LANES = 16


def _gather_body(table_hbm, idxt_hbm, out_hbm, idx_v, gb_a, gb_b, tb_a, tb_b, *sems):
    n_hist = idxt_hbm.shape[0]
    gbuf = (gb_a, gb_b)
    tbuf = (tb_a, tb_b)
    gsem = sems[0:2]
    wsem = sems[2:4]

    wid = lax.axis_index("s") * 2 + lax.axis_index("c")
    pltpu.sync_copy(idxt_hbm.at[:, pl.ds(wid * CHUNK, CHUNK)], idx_v)

    def gather(h, p):
        return pltpu.make_async_copy(table_hbm.at[idx_v.at[h]], gbuf[p], gsem[p])

    def write(h, a, p):
        dst = out_hbm.at[pl.ds(h * (8 * NUM_WORKERS * 8) + a * (NUM_WORKERS * 8) + wid * 8, 8)]
        return pltpu.make_async_copy(tbuf[p].at[pl.ds(a * 8, 8)], dst, wsem[p])

    rows_g = [jnp.arange(LANES, dtype=jnp.int32) + g * LANES for g in range(CHUNK // LANES)]

    def transpose(p):
        def fbody(f, carry):
            cols = jnp.full((LANES,), f, dtype=jnp.int32)
            for g in range(CHUNK // LANES):
                v = plsc.load_gather(gbuf[p], [rows_g[g], cols])
                tbuf[p][f, pl.ds(g * LANES, LANES)] = v
            return carry
        lax.fori_loop(0, DIM, fbody, 0)

    def visit(h, p, first):
        gather(h, p).wait()
        if not first:
            for a in range(8):
                write(0, a, p).wait()
        transpose(p)
        gather(h + 2, p).start()
        for a in range(8):
            write(h, a, p).start()

    gather(0, 0).start()
    gather(1, 1).start()
    visit(0, 0, True)
    visit(1, 1, True)

    def body(k, carry):
        for p in (0, 1):
            visit(2 * k + p, p, False)
        return carry

    lax.fori_loop(1, n_hist // 2 - 1, body, 0)

    for h in (n_hist - 2, n_hist - 1):
        p = h % 2
        gather(h, p).wait()
        for a in range(8):
            write(0, a, p).wait()
        transpose(p)
        for a in range(8):
            write(h, a, p).start()
    for p in (0, 1):
        for a in range(8):
            write(0, a, p).wait()


@jax.jit
def _run(idxt, table_pad):
    n_total = idxt.shape[0] * idxt.shape[1]
    mesh = plsc.VectorSubcoreMesh(core_axis_name="c", subcore_axis_name="s")
    k = functools.partial(
        pl.kernel,
        mesh=mesh,
        compiler_params=pltpu.CompilerParams(use_tc_tiling_on_sc=False, needs_layout_passes=False),
        out_type=jax.ShapeDtypeStruct((n_total // 2, 2 * DIM), jnp.float32),
        scratch_types=[pltpu.VMEM((idxt.shape[0], CHUNK), jnp.int32)]
        + [pltpu.VMEM((CHUNK, 2 * DIM), jnp.float32) for _ in range(2)]
        + [pltpu.VMEM((DIM, CHUNK), jnp.float32) for _ in range(2)]
        + [pltpu.SemaphoreType.DMA for _ in range(4)],
    )(_gather_body)
    return k(table_pad, idxt)


def kernel(input_data, embeddings_matrix):
    b, h = input_data.shape
    idxt = input_data.astype(jnp.int32).T
    table_pad = jnp.pad(embeddings_matrix, ((0, 0), (0, 2 * DIM - embeddings_matrix.shape[1])))
    out = _run(idxt, table_pad)
    return (
        out.reshape(h, 8, NUM_WORKERS, 8, CHUNK)
        .transpose(2, 4, 0, 1, 3)
        .reshape(b, h, DIM)
    )


# speedup vs baseline: 1.0000x; 1.0000x over previous
import functools

import jax
import jax.numpy as jnp
from jax import lax
from jax.experimental import pallas as pl
from jax.experimental.pallas import tpu as pltpu
from jax.experimental.pallas import tpu_sc as plsc

DIM = 64
NUM_WORKERS = 32
CHUNK = 128
LANES = 16


def _gather_body(table_hbm, idxt_hbm, out_hbm, idx_v, gb_a, gb_b, tb_a, tb_b, *sems):
    n_hist = idxt_hbm.shape[0]
    gbuf = (gb_a, gb_b)
    tbuf = (tb_a, tb_b)
    gsem = sems[0:2]
    wsem = sems[2:4]

    wid = lax.axis_index("s") * 2 + lax.axis_index("c")
    pltpu.sync_copy(idxt_hbm.at[:, pl.ds(wid * CHUNK, CHUNK)], idx_v)

    def gather(h, p):
        return pltpu.make_async_copy(table_hbm.at[idx_v.at[h]], gbuf[p], gsem[p])

    def write(h, a, p):
        dst = out_hbm.at[pl.ds(h * (8 * NUM_WORKERS * 8) + a * (NUM_WORKERS * 8) + wid * 8, 8)]
        return pltpu.make_async_copy(tbuf[p].at[pl.ds(a * 8, 8)], dst, wsem[p])

    rows_g = [jnp.arange(LANES, dtype=jnp.int32) + g * LANES for g in range(CHUNK // LANES)]

    def transpose(p):
        def fbody(k, carry):
            for df in range(4):
                f = k * 4 + df
                cols = jnp.full((LANES,), f, dtype=jnp.int32)
                for g in range(CHUNK // LANES):
                    v = plsc.load_gather(gbuf[p], [rows_g[g], cols])
                    tbuf[p][f, pl.ds(g * LANES, LANES)] = v
            return carry
        lax.fori_loop(0, DIM // 4, fbody, 0)

    def visit(h, p, first):
        gather(h, p).wait()
        if not first:
            for a in range(8):
                write(0, a, p).wait()
        transpose(p)
        gather(h + 2, p).start()
        for a in range(8):
            write(h, a, p).start()

    gather(0, 0).start()
    gather(1, 1).start()
    visit(0, 0, True)
    visit(1, 1, True)

    def body(k, carry):
        for p in (0, 1):
            visit(2 * k + p, p, False)
        return carry

    lax.fori_loop(1, n_hist // 2 - 1, body, 0)

    for h in (n_hist - 2, n_hist - 1):
        p = h % 2
        gather(h, p).wait()
        for a in range(8):
            write(0, a, p).wait()
        transpose(p)
        for a in range(8):
            write(h, a, p).start()
    for p in (0, 1):
        for a in range(8):
            write(0, a, p).wait()


@jax.jit
def _run(idxt, table_pad):
    n_total = idxt.shape[0] * idxt.shape[1]
    mesh = plsc.VectorSubcoreMesh(core_axis_name="c", subcore_axis_name="s")
    k = functools.partial(
        pl.kernel,
        mesh=mesh,
        compiler_params=pltpu.CompilerParams(use_tc_tiling_on_sc=False, needs_layout_passes=False),
        out_type=jax.ShapeDtypeStruct((n_total // 2, 2 * DIM), jnp.float32),
        scratch_types=[pltpu.VMEM((idxt.shape[0], CHUNK), jnp.int32)]
        + [pltpu.VMEM((CHUNK, 2 * DIM), jnp.float32) for _ in range(2)]
        + [pltpu.VMEM((DIM, CHUNK), jnp.float32) for _ in range(2)]
        + [pltpu.SemaphoreType.DMA for _ in range(4)],
    )(_gather_body)
    return k(table_pad, idxt)


def kernel(input_data, embeddings_matrix):
    b, h = input_data.shape
    idxt = input_data.astype(jnp.int32).T
    table_pad = jnp.pad(embeddings_matrix, ((0, 0), (0, 2 * DIM - embeddings_matrix.shape[1])))
    out = _run(idxt, table_pad)
    return (
        out.reshape(h, 8, NUM_WORKERS, 8, CHUNK)
        .transpose(2, 4, 0, 1, 3)
        .reshape(b, h, DIM)
    )


# speedup vs baseline: 1.1577x; 1.1577x over previous
import functools

import jax
import jax.numpy as jnp
from jax import lax
from jax.experimental import pallas as pl
from jax.experimental.pallas import tpu as pltpu
from jax.experimental.pallas import tpu_sc as plsc

DIM = 64
NUM_WORKERS = 32
CHUNK = 128
LANES = 16


def _gather_body(table_hbm, idx_hbm, out_hbm, idx_v, gb_a, gb_b, *sems):
    n_chunk = idx_hbm.shape[0] // NUM_WORKERS
    gbuf = (gb_a, gb_b)
    gsem = sems[0:2]
    wsem = sems[2:4]

    wid = lax.axis_index("s") * 2 + lax.axis_index("c")
    base = wid * (n_chunk * CHUNK)
    pltpu.sync_copy(idx_hbm.at[pl.ds(wid * n_chunk, n_chunk)], idx_v)

    def gather(j, p):
        return pltpu.make_async_copy(table_hbm.at[idx_v.at[j]], gbuf[p], gsem[p])

    def write(j, p):
        dst = out_hbm.at[pl.ds(base + j * CHUNK, CHUNK)]
        return pltpu.make_async_copy(gbuf[p].at[:, pl.ds(0, DIM)], dst, wsem[p])

    def visit(j, p, first):
        gather(j, p).wait()
        write(j, p).start()
        write(0, p).wait()
        if True:
            gather(j + 2, p).start()

    gather(0, 0).start()
    gather(1, 1).start()
    visit(0, 0, True)
    visit(1, 1, True)

    def body(k, carry):
        for p in (0, 1):
            visit(2 * k + p, p, False)
        return carry

    lax.fori_loop(1, n_chunk // 2 - 1, body, 0)

    for j in (n_chunk - 2, n_chunk - 1):
        p = j % 2
        gather(j, p).wait()
        write(j, p).start()
        write(0, p).wait()


@jax.jit
def _run(idx, table_pad):
    n_total = idx.shape[0] * idx.shape[1]
    mesh = plsc.VectorSubcoreMesh(core_axis_name="c", subcore_axis_name="s")
    k = functools.partial(
        pl.kernel,
        mesh=mesh,
        compiler_params=pltpu.CompilerParams(use_tc_tiling_on_sc=False),
        out_type=jax.ShapeDtypeStruct((n_total, DIM), jnp.float32),
        scratch_types=[pltpu.VMEM((idx.shape[0] // NUM_WORKERS, CHUNK), jnp.int32)]
        + [pltpu.VMEM((CHUNK, 2 * DIM), jnp.float32) for _ in range(2)]
        + [pltpu.SemaphoreType.DMA for _ in range(4)],
    )(_gather_body)
    return k(table_pad, idx)


def kernel(input_data, embeddings_matrix):
    b, h = input_data.shape
    idx = input_data.astype(jnp.int32).reshape(b * h // CHUNK, CHUNK)
    table_pad = jnp.pad(embeddings_matrix, ((0, 0), (0, 2 * DIM - embeddings_matrix.shape[1])))
    out = _run(idx, table_pad)
    return out.reshape(b, h, DIM)
